# baseline (speedup 1.0000x reference)
import jax
import jax.numpy as jnp
from jax.experimental import pallas as pl
from jax.experimental.pallas import tpu as pltpu


def _dm_kernel(ids_s,
               para_hbm,
               word3, outT3,
               o_ref,
               pbuf,
               wsum,
               psem):
    S, TB = o_ref.shape
    C = ids_s.shape[1] - 1 - S
    base = pl.program_id(0) * TB

    lane = jax.lax.broadcasted_iota(jnp.int32, (S, TB), 1)

    for k in range(TB):
        pltpu.make_async_copy(para_hbm.at[ids_s[base + k, 0]],
                              pbuf.at[k], psem).start()

    def _tree_sum(vals):
        while len(vals) > 1:
            nxt = [a + b for a, b in zip(vals[::2], vals[1::2])]
            if len(vals) % 2:
                nxt.append(vals[-1])
            vals = nxt
        return vals[0]

    for r in range(TB):
        wsum[r] = _tree_sum([word3[ids_s[base + r, 1 + c]]
                             for c in range(C)])

    pltpu.make_async_copy(pbuf, pbuf, psem).wait()

    terms = []
    for r in range(TB):
        gs = jnp.concatenate([outT3[ids_s[base + r, 1 + C + s]]
                              for s in range(S)], axis=0)
        p = (wsum[r] + pbuf[r]) * gs
        col = jnp.sum(p, axis=1, keepdims=True)
        terms.append(jnp.where(lane == r, col, 0.0))
    o_ref[...] = _tree_sum(terms)


def kernel(doc_ids, context_ids, sample_ids, paragraph_matrix, word_matrix,
           outputs):
    B, C = context_ids.shape
    S = sample_ids.shape[1]
    n_docs, D = paragraph_matrix.shape
    n_words = word_matrix.shape[0]

    TB = 128 if B % 128 == 0 else 8
    B_pad = ((B + TB - 1) // TB) * TB

    pad_b = B_pad - B
    ids = jnp.concatenate([doc_ids.astype(jnp.int32).reshape(B, 1),
                           context_ids.astype(jnp.int32),
                           sample_ids.astype(jnp.int32)], axis=1)
    if pad_b:
        ids = jnp.pad(ids, ((0, pad_b), (0, 0)))

    para3 = paragraph_matrix.reshape(n_docs, 1, D)
    word3 = word_matrix.reshape(n_words, 1, D)
    outT3 = jnp.swapaxes(outputs, 0, 1).reshape(n_words, 1, D)

    res = pl.pallas_call(
        _dm_kernel,
        grid=(B_pad // TB,),
        in_specs=[
            pl.BlockSpec((B_pad, 1 + C + S), lambda i: (0, 0),
                         memory_space=pltpu.SMEM),
            pl.BlockSpec(memory_space=pl.ANY),
            pl.BlockSpec((n_words, 1, D), lambda i: (0, 0, 0)),
            pl.BlockSpec((n_words, 1, D), lambda i: (0, 0, 0)),
        ],
        out_specs=pl.BlockSpec((S, TB), lambda i: (0, i)),
        out_shape=jax.ShapeDtypeStruct((S, B_pad), jnp.float32),
        scratch_shapes=[pltpu.VMEM((TB, 1, D), jnp.float32),
                        pltpu.VMEM((TB, 1, D), jnp.float32),
                        pltpu.SemaphoreType.DMA],
        compiler_params=pltpu.CompilerParams(
            dimension_semantics=("parallel",),
            allow_input_fusion=[True, True, True, True],
            vmem_limit_bytes=64 * 1024 * 1024),
    )(ids, para3, word3, outT3)

    return jnp.squeeze(res[:, :B].T)


# speedup vs baseline: 1.0015x; 1.0015x over previous
import jax
import jax.numpy as jnp
from jax.experimental import pallas as pl
from jax.experimental.pallas import tpu as pltpu


def _dm_kernel(ids_s,
               para_hbm,
               word3, outT3,
               o_ref,
               pbuf,
               wsum,
               psem):
    S, TB = o_ref.shape
    C = ids_s.shape[1] - 1 - S
    base = pl.program_id(0) * TB

    lane = jax.lax.broadcasted_iota(jnp.int32, (S, TB), 1)

    for k in range(TB):
        pltpu.make_async_copy(para_hbm.at[ids_s[base + k, 0]],
                              pbuf.at[k], psem).start()

    def _tree_sum(vals):
        while len(vals) > 1:
            nxt = [a + b for a, b in zip(vals[::2], vals[1::2])]
            if len(vals) % 2:
                nxt.append(vals[-1])
            vals = nxt
        return vals[0]

    for r in range(TB):
        wsum[r] = _tree_sum([word3[ids_s[base + r, 1 + c]]
                             for c in range(C)])

    pltpu.make_async_copy(pbuf, pbuf, psem).wait()

    terms = []
    for r in range(TB):
        gs = jnp.concatenate([outT3[ids_s[base + r, 1 + C + s]]
                              for s in range(S)], axis=0)
        p = (wsum[r] + pbuf[r]) * gs
        col = jnp.sum(p, axis=1, keepdims=True)
        terms.append(jnp.where(lane == r, col, 0.0))
    o_ref[...] = _tree_sum(terms)


def kernel(doc_ids, context_ids, sample_ids, paragraph_matrix, word_matrix,
           outputs):
    B, C = context_ids.shape
    S = sample_ids.shape[1]
    n_docs, D = paragraph_matrix.shape
    n_words = word_matrix.shape[0]

    TB = 128 if B % 128 == 0 else 8
    B_pad = ((B + TB - 1) // TB) * TB

    pad_b = B_pad - B
    ids = jnp.concatenate([doc_ids.astype(jnp.int32).reshape(B, 1),
                           context_ids.astype(jnp.int32),
                           sample_ids.astype(jnp.int32)], axis=1)
    if pad_b:
        ids = jnp.pad(ids, ((0, pad_b), (0, 0)))

    para3 = paragraph_matrix.reshape(n_docs, 1, D)
    word3 = word_matrix.reshape(n_words, 1, D)
    outT3 = jnp.swapaxes(outputs, 0, 1).reshape(n_words, 1, D)

    res = pl.pallas_call(
        _dm_kernel,
        grid=(B_pad // TB,),
        in_specs=[
            pl.BlockSpec((B_pad, 1 + C + S), lambda i: (0, 0),
                         memory_space=pltpu.SMEM),
            pl.BlockSpec(memory_space=pl.ANY),
            pl.BlockSpec((n_words, 1, D), lambda i: (0, 0, 0)),
            pl.BlockSpec((n_words, 1, D), lambda i: (0, 0, 0)),
        ],
        out_specs=pl.BlockSpec((S, TB), lambda i: (0, i)),
        out_shape=jax.ShapeDtypeStruct((S, B_pad), jnp.float32),
        scratch_shapes=[pltpu.VMEM((TB, 1, D), jnp.float32),
                        pltpu.VMEM((TB, 1, D), jnp.float32),
                        pltpu.SemaphoreType.DMA],
        compiler_params=pltpu.CompilerParams(
            dimension_semantics=("parallel",),
            vmem_limit_bytes=64 * 1024 * 1024),
    )(ids, para3, word3, outT3)

    return jnp.squeeze(res[:, :B].T)
